# Initial kernel scaffold; baseline (speedup 1.0000x reference)
import jax
import jax.numpy as jnp
from jax.experimental import pallas as pl


def kernel(x, edge_index, batch, rdkit_feats, W1, b1, gamma1, beta1, W2, b2, gamma2, beta2, W3, b3, gamma3, beta3, W4, b4, gamma4, beta4, W5, b5):
    raise NotImplementedError("write your pallas kernel here")


# speedup vs baseline: 9.5796x; 9.5796x over previous
import functools

import jax
import jax.numpy as jnp
from jax import lax
from jax.experimental import pallas as pl
from jax.experimental.pallas import tpu as pltpu
from jax.experimental.pallas import tpu_sc as plsc

F32 = jnp.float32
I32 = jnp.int32

N = 10000
E = 320000
D = 128
B = 256
R = 200
H = 128

NP = 10240
NC, NS = 2, 16
NW = NC * NS
CHUNKS = 79
K = 128
EPW = CHUNKS * K
EP = EPW * NW
PAD_ROW = NP - 1
DW = 128
ZR = NP // NS

PRW = NP // NW
PC, PK = 5, 64
SEG = 384
PAD_SEG = SEG - 1

BR = 512
GRID = NP // BR
EPS = 1e-5


def _sc_mesh():
    return plsc.VectorSubcoreMesh(
        core_axis_name="c", subcore_axis_name="s",
        num_cores=NC, num_subcores=NS)


@functools.cache
def _deg_kernel():
    @functools.partial(
        pl.kernel,
        out_type=jax.ShapeDtypeStruct((NC, NP, DW), F32),
        mesh=_sc_mesh(),
        scratch_types=[
            pltpu.VMEM((K,), I32),
            pltpu.VMEM((16, DW), F32),
            pltpu.VMEM_SHARED((NP, DW), F32),
        ],
    )
    def deg(dsts_hbm, zeros_hbm, ones_hbm, out_hbm, idx_d, ones_v, acc):
        c = lax.axis_index("c")
        s = lax.axis_index("s")
        pltpu.sync_copy(zeros_hbm, acc.at[pl.ds(s * ZR, ZR)])
        pltpu.sync_copy(ones_hbm, ones_v)
        plsc.subcore_barrier()

        def chunk(j, carry):
            pltpu.sync_copy(dsts_hbm.at[c, s, j], idx_d)
            for t in range(K // 16):
                iv = idx_d[pl.ds(t * 16, 16)]
                pltpu.sync_copy(ones_v, acc.at[iv], add=True)
            return carry

        lax.fori_loop(0, CHUNKS, chunk, 0)
        plsc.subcore_barrier()
        pltpu.sync_copy(acc.at[pl.ds(s * ZR, ZR)],
                        out_hbm.at[c, pl.ds(s * ZR, ZR)])

    return deg


@functools.cache
def _agg_kernel():
    @functools.partial(
        pl.kernel,
        out_type=jax.ShapeDtypeStruct((NC, NP, D), F32),
        mesh=_sc_mesh(),
        scratch_types=[
            pltpu.VMEM((K,), I32),
            pltpu.VMEM((K,), I32),
            pltpu.VMEM((K, D), F32),
            pltpu.VMEM_SHARED((NP, D), F32),
            pltpu.SemaphoreType.DMA,
        ],
    )
    def agg(g_hbm, srcs_hbm, dsts_hbm, zeros_hbm, out_hbm,
            idx_s, idx_d, rows, acc, sem):
        c = lax.axis_index("c")
        s = lax.axis_index("s")
        pltpu.sync_copy(zeros_hbm, acc.at[pl.ds(s * ZR, ZR)])
        plsc.subcore_barrier()

        def chunk(j, carry):
            pltpu.sync_copy(srcs_hbm.at[c, s, j], idx_s)
            pltpu.sync_copy(dsts_hbm.at[c, s, j], idx_d)
            pltpu.async_copy(g_hbm.at[idx_s], rows, sem).wait()
            for t in range(K // 16):
                iv = idx_d[pl.ds(t * 16, 16)]
                pltpu.sync_copy(rows.at[pl.ds(t * 16, 16)], acc.at[iv],
                                add=True)
            return carry

        lax.fori_loop(0, CHUNKS, chunk, 0)
        plsc.subcore_barrier()
        pltpu.sync_copy(acc.at[pl.ds(s * ZR, ZR)],
                        out_hbm.at[c, pl.ds(s * ZR, ZR)])

    return agg


@functools.cache
def _pool_kernel():
    @functools.partial(
        pl.kernel,
        out_type=(jax.ShapeDtypeStruct((NC, SEG, H), F32),
                  jax.ShapeDtypeStruct((NC, SEG, H), F32),
                  jax.ShapeDtypeStruct((NC, SEG, DW), F32)),
        mesh=_sc_mesh(),
        scratch_types=[
            pltpu.VMEM((PK,), I32),
            pltpu.VMEM((PK, H), F32),
            pltpu.VMEM((PK, H), F32),
            pltpu.VMEM((16, DW), F32),
            pltpu.VMEM_SHARED((SEG, H), F32),
            pltpu.VMEM_SHARED((SEG, H), F32),
            pltpu.VMEM_SHARED((SEG, DW), F32),
        ],
    )
    def pool(h2a_hbm, h2b_hbm, batch_hbm, zseg_hbm, zcnt_hbm, ones_hbm,
             sega_hbm, segb_hbm, cnt_hbm, idx_b, rows_a, rows_b, ones_v,
             acc_a, acc_b, acc_cnt):
        c = lax.axis_index("c")
        s = lax.axis_index("s")
        zr = SEG // NS
        pltpu.sync_copy(zseg_hbm, acc_a.at[pl.ds(s * zr, zr)])
        pltpu.sync_copy(zseg_hbm, acc_b.at[pl.ds(s * zr, zr)])
        pltpu.sync_copy(zcnt_hbm, acc_cnt.at[pl.ds(s * zr, zr)])
        pltpu.sync_copy(ones_hbm, ones_v)
        plsc.subcore_barrier()
        base = (c * NS + s) * PRW

        def chunk(j, carry):
            pltpu.sync_copy(batch_hbm.at[c, s, j], idx_b)
            pltpu.sync_copy(h2a_hbm.at[pl.ds(base + j * PK, PK)], rows_a)
            pltpu.sync_copy(h2b_hbm.at[pl.ds(base + j * PK, PK)], rows_b)
            for t in range(PK // 16):
                iv = idx_b[pl.ds(t * 16, 16)]
                pltpu.sync_copy(rows_a.at[pl.ds(t * 16, 16)], acc_a.at[iv],
                                add=True)
                pltpu.sync_copy(rows_b.at[pl.ds(t * 16, 16)], acc_b.at[iv],
                                add=True)
                pltpu.sync_copy(ones_v, acc_cnt.at[iv], add=True)
            return carry

        lax.fori_loop(0, PC, chunk, 0)
        plsc.subcore_barrier()
        pltpu.sync_copy(acc_a.at[pl.ds(s * zr, zr)],
                        sega_hbm.at[c, pl.ds(s * zr, zr)])
        pltpu.sync_copy(acc_b.at[pl.ds(s * zr, zr)],
                        segb_hbm.at[c, pl.ds(s * zr, zr)])
        pltpu.sync_copy(acc_cnt.at[pl.ds(s * zr, zr)],
                        cnt_hbm.at[c, pl.ds(s * zr, zr)])

    return pool


def _row_spec(w):
    return pl.BlockSpec((BR, w), lambda i: (i, 0))


def _whole(shape):
    return pl.BlockSpec(shape, lambda i: tuple(0 for _ in shape))


def _prep_body(x_ref, w1_ref, d0_ref, d1_ref, g1_ref, dinv_ref):
    d = d0_ref[:, 0:1] + d1_ref[:, 0:1] + 1.0
    dinv = jnp.broadcast_to(lax.rsqrt(d), (BR, D))
    xw = jnp.dot(x_ref[...], w1_ref[...], preferred_element_type=F32)
    g1_ref[...] = xw * dinv
    dinv_ref[...] = dinv


def _tc_prep(x_p, w1, deg0, deg1):
    return pl.pallas_call(
        _prep_body,
        grid=(GRID,),
        in_specs=[_row_spec(D), _whole((D, H)), _row_spec(DW), _row_spec(DW)],
        out_specs=[_row_spec(H), _row_spec(D)],
        out_shape=[jax.ShapeDtypeStruct((NP, H), F32),
                   jax.ShapeDtypeStruct((NP, D), F32)],
    )(x_p, w1, deg0, deg1)


def _epi1_body(s0_ref, s1_ref, g1_ref, dinv_ref, b1_ref,
               r1_ref, sum_ref, sq_ref):
    i = pl.program_id(0)
    r = jnp.maximum(
        dinv_ref[...] * (s0_ref[...] + s1_ref[...] + g1_ref[...])
        + b1_ref[...], 0.0)
    r1_ref[...] = r
    rowid = lax.broadcasted_iota(I32, (BR, H), 0) + i * BR
    rm = jnp.where(rowid < N, r, 0.0)

    @pl.when(i == 0)
    def _():
        sum_ref[...] = jnp.zeros((8, H), F32)
        sq_ref[...] = jnp.zeros((8, H), F32)

    sum_ref[0:1, :] += jnp.sum(rm, axis=0, keepdims=True)
    sq_ref[0:1, :] += jnp.sum(rm * rm, axis=0, keepdims=True)


def _tc_epi1(s1_parts, g1, dinv_col, b1):
    return pl.pallas_call(
        _epi1_body,
        grid=(GRID,),
        in_specs=[_row_spec(H), _row_spec(H), _row_spec(H), _row_spec(D),
                  _whole((1, H))],
        out_specs=[_row_spec(H), _whole((8, H)), _whole((8, H))],
        out_shape=[jax.ShapeDtypeStruct((NP, H), F32),
                   jax.ShapeDtypeStruct((8, H), F32),
                   jax.ShapeDtypeStruct((8, H), F32)],
    )(s1_parts[0], s1_parts[1], g1, dinv_col, b1)


def _bnmm_body(r1_ref, sum_ref, sq_ref, ga_ref, be_ref, w2a_ref, w2b_ref,
               dinv_ref, g2a_ref, g2b_ref):
    mu = sum_ref[0:1, :] * (1.0 / N)
    var = sq_ref[0:1, :] * (1.0 / N) - mu * mu
    sfac = ga_ref[...] * lax.rsqrt(var + EPS)
    t = be_ref[...] - mu * sfac
    h = r1_ref[...] * sfac + t
    dv = dinv_ref[...]
    g2a_ref[...] = jnp.dot(h, w2a_ref[...], preferred_element_type=F32) * dv
    g2b_ref[...] = jnp.dot(h, w2b_ref[...], preferred_element_type=F32) * dv


def _tc_bnmm(r1, ssum, ssq, gamma1, beta1, w2a, w2b, dinv_col):
    return pl.pallas_call(
        _bnmm_body,
        grid=(GRID,),
        in_specs=[_row_spec(H), _whole((8, H)), _whole((8, H)),
                  _whole((1, H)), _whole((1, H)),
                  _whole((H, H)), _whole((H, H)), _row_spec(D)],
        out_specs=[_row_spec(H), _row_spec(H)],
        out_shape=[jax.ShapeDtypeStruct((NP, H), F32),
                   jax.ShapeDtypeStruct((NP, H), F32)],
    )(r1, ssum, ssq, gamma1, beta1, w2a, w2b, dinv_col)


def _epi2_body(sa0_ref, sa1_ref, sb0_ref, sb1_ref, g2a_ref, g2b_ref,
               dinv_ref, b2a_ref, b2b_ref, r2_ref, sum_ref, sq_ref):
    i = pl.program_id(0)
    dv = dinv_ref[...]
    ra = jnp.maximum(
        dv * (sa0_ref[...] + sa1_ref[...] + g2a_ref[...]) + b2a_ref[...], 0.0)
    rb = jnp.maximum(
        dv * (sb0_ref[...] + sb1_ref[...] + g2b_ref[...]) + b2b_ref[...], 0.0)
    r2_ref[:, 0:H] = ra
    r2_ref[:, H:2 * H] = rb
    rowid = lax.broadcasted_iota(I32, (BR, H), 0) + i * BR
    ram = jnp.where(rowid < N, ra, 0.0)
    rbm = jnp.where(rowid < N, rb, 0.0)

    @pl.when(i == 0)
    def _():
        sum_ref[...] = jnp.zeros((8, 2 * H), F32)
        sq_ref[...] = jnp.zeros((8, 2 * H), F32)

    sum_ref[0:1, 0:H] += jnp.sum(ram, axis=0, keepdims=True)
    sum_ref[0:1, H:2 * H] += jnp.sum(rbm, axis=0, keepdims=True)
    sq_ref[0:1, 0:H] += jnp.sum(ram * ram, axis=0, keepdims=True)
    sq_ref[0:1, H:2 * H] += jnp.sum(rbm * rbm, axis=0, keepdims=True)


def _tc_epi2(s2a, s2b, g2a, g2b, dinv_col, b2a, b2b):
    return pl.pallas_call(
        _epi2_body,
        grid=(GRID,),
        in_specs=[_row_spec(H), _row_spec(H), _row_spec(H), _row_spec(H),
                  _row_spec(H), _row_spec(H), _row_spec(D),
                  _whole((1, H)), _whole((1, H))],
        out_specs=[_row_spec(2 * H), _whole((8, 2 * H)), _whole((8, 2 * H))],
        out_shape=[jax.ShapeDtypeStruct((NP, 2 * H), F32),
                   jax.ShapeDtypeStruct((8, 2 * H), F32),
                   jax.ShapeDtypeStruct((8, 2 * H), F32)],
    )(s2a[0], s2a[1], s2b[0], s2b[1], g2a, g2b, dinv_col, b2a, b2b)


def _bn2_body(r2_ref, sum_ref, sq_ref, ga_ref, be_ref, h2a_ref, h2b_ref):
    mu = sum_ref[0:1, :] * (1.0 / N)
    var = sq_ref[0:1, :] * (1.0 / N) - mu * mu
    sfac = ga_ref[...] * lax.rsqrt(var + EPS)
    t = be_ref[...] - mu * sfac
    h2 = r2_ref[...] * sfac + t
    h2a_ref[...] = h2[:, 0:H]
    h2b_ref[...] = h2[:, H:2 * H]


def _tc_bn2(r2, ssum, ssq, gamma2, beta2):
    return pl.pallas_call(
        _bn2_body,
        grid=(GRID,),
        in_specs=[_row_spec(2 * H), _whole((8, 2 * H)), _whole((8, 2 * H)),
                  _whole((1, 2 * H)), _whole((1, 2 * H))],
        out_specs=[_row_spec(H), _row_spec(H)],
        out_shape=[jax.ShapeDtypeStruct((NP, H), F32),
                   jax.ShapeDtypeStruct((NP, H), F32)],
    )(r2, ssum, ssq, gamma2, beta2)


def _bn_rows(z, gamma, beta):
    mu = jnp.mean(z, axis=0, keepdims=True)
    var = jnp.mean(z * z, axis=0, keepdims=True) - mu * mu
    return (z - mu) * lax.rsqrt(var + EPS) * gamma + beta


def _head_body(s0a_ref, s0b_ref, s1a_ref, s1b_ref, cnt0_ref, cnt1_ref,
               rdk_ref, w3ea_ref, w3eb_ref, w3r_ref, b3_ref, g3_ref, be3_ref,
               w4_ref, b4_ref, g4_ref, be4_ref, w5_ref, b5_ref, out_ref):
    cnt = cnt0_ref[:, 0:1] + cnt1_ref[:, 0:1]
    inv = 1.0 / jnp.maximum(cnt, 1.0)
    emb_a = (s0a_ref[...] + s1a_ref[...]) * inv
    emb_b = (s0b_ref[...] + s1b_ref[...]) * inv
    z = jnp.dot(emb_a, w3ea_ref[...], preferred_element_type=F32)
    z += jnp.dot(emb_b, w3eb_ref[...], preferred_element_type=F32)
    z += jnp.dot(rdk_ref[...], w3r_ref[...], preferred_element_type=F32)
    z = jnp.maximum(z + b3_ref[...], 0.0)
    z = _bn_rows(z, g3_ref[...], be3_ref[...])
    z = jnp.maximum(
        jnp.dot(z, w4_ref[...], preferred_element_type=F32) + b4_ref[...], 0.0)
    z = _bn_rows(z, g4_ref[...], be4_ref[...])
    out_ref[...] = (jnp.dot(z, w5_ref[...], preferred_element_type=F32)
                    + b5_ref[...])


def _tc_head(s0a, s0b, s1a, s1b, cnt0, cnt1, rdk_p, w3ea, w3eb, w3r,
             b3, g3, be3, w4, b4, g4, be4, w5_p, b5_p):
    shapes = [(B, H), (B, H), (B, H), (B, H), (B, DW), (B, DW), (B, 2 * H),
              (H, 2 * H), (H, 2 * H), (2 * H, 2 * H), (1, 2 * H), (1, 2 * H),
              (1, 2 * H), (2 * H, H), (1, H), (1, H), (1, H),
              (H, H), (1, H)]
    return pl.pallas_call(
        _head_body,
        grid=(1,),
        in_specs=[_whole(s) for s in shapes],
        out_specs=_whole((B, H)),
        out_shape=jax.ShapeDtypeStruct((B, H), F32),
    )(s0a, s0b, s1a, s1b, cnt0, cnt1, rdk_p, w3ea, w3eb, w3r,
      b3, g3, be3, w4, b4, g4, be4, w5_p, b5_p)


def kernel(x, edge_index, batch, rdkit_feats, W1, b1, gamma1, beta1,
           W2, b2, gamma2, beta2, W3, b3, gamma3, beta3,
           W4, b4, gamma4, beta4, W5, b5):
    ei = edge_index.astype(I32)
    pad_e = N + (jnp.arange(EP - E, dtype=I32) % (NP - N))
    srcs = jnp.concatenate([ei[0], pad_e]).reshape(NC, NS, CHUNKS, K)
    dsts = jnp.concatenate([ei[1], pad_e]).reshape(NC, NS, CHUNKS, K)
    x_p = jnp.pad(x, ((0, NP - N), (0, 0)))
    batch_p = jnp.concatenate(
        [batch.astype(I32),
         B + (jnp.arange(NP - N, dtype=I32) % (SEG - B))]
    ).reshape(NC, NS, PC, PK)
    rdk_p = jnp.pad(rdkit_feats, ((0, 0), (0, 2 * H - R)))

    zeros_deg = jnp.zeros((ZR, DW), F32)
    ones_deg = jnp.ones((16, DW), F32)
    zeros_agg = jnp.zeros((ZR, D), F32)
    zseg = jnp.zeros((SEG // NS, H), F32)
    zcnt = jnp.zeros((SEG // NS, DW), F32)
    ones_pool = jnp.ones((16, DW), F32)

    b1r = b1.reshape(1, H)
    g1r, be1r = gamma1.reshape(1, H), beta1.reshape(1, H)
    w2a, w2b = W2[:, :H], W2[:, H:]
    b2a, b2b = b2[:H].reshape(1, H), b2[H:].reshape(1, H)
    g2r, be2r = gamma2.reshape(1, 2 * H), beta2.reshape(1, 2 * H)
    w3ea, w3eb = W3[:H], W3[H:2 * H]
    w3r = jnp.pad(W3[2 * H:], ((0, 2 * H - R), (0, 0)))
    b3r = b3.reshape(1, 2 * H)
    g3r, be3r = gamma3.reshape(1, 2 * H), beta3.reshape(1, 2 * H)
    b4r = b4.reshape(1, H)
    g4r, be4r = gamma4.reshape(1, H), beta4.reshape(1, H)
    w5_p = jnp.pad(W5, ((0, 0), (0, H - 1)))
    b5_p = jnp.pad(b5, (0, H - 1)).reshape(1, H)

    deg = _deg_kernel()(dsts, zeros_deg, ones_deg)
    g1, dinv_col = _tc_prep(x_p, W1, deg[0], deg[1])

    s1 = _agg_kernel()(g1, srcs, dsts, zeros_agg)
    r1, ssum1, ssq1 = _tc_epi1((s1[0], s1[1]), g1, dinv_col, b1r)

    g2a, g2b = _tc_bnmm(r1, ssum1, ssq1, g1r, be1r, w2a, w2b, dinv_col)

    s2a = _agg_kernel()(g2a, srcs, dsts, zeros_agg)
    s2b = _agg_kernel()(g2b, srcs, dsts, zeros_agg)
    r2, ssum2, ssq2 = _tc_epi2((s2a[0], s2a[1]), (s2b[0], s2b[1]),
                               g2a, g2b, dinv_col, b2a, b2b)
    h2a, h2b = _tc_bn2(r2, ssum2, ssq2, g2r, be2r)

    seg_a, seg_b, cnt = _pool_kernel()(h2a, h2b, batch_p, zseg, zcnt,
                                       ones_pool)

    out = _tc_head(seg_a[0, :B], seg_b[0, :B], seg_a[1, :B], seg_b[1, :B],
                   cnt[0, :B], cnt[1, :B], rdk_p, w3ea, w3eb, w3r,
                   b3r, g3r, be3r, W4, b4r, g4r, be4r, w5_p, b5_p)
    return out[:, 0]


# speedup vs baseline: 14.6743x; 1.5318x over previous
import functools

import jax
import jax.numpy as jnp
from jax import lax
from jax.experimental import pallas as pl
from jax.experimental.pallas import tpu as pltpu
from jax.experimental.pallas import tpu_sc as plsc

F32 = jnp.float32
I32 = jnp.int32

N = 10000
E = 320000
D = 128
B = 256
R = 200
H = 128

NP = 10240
NC, NS = 2, 16
NW = NC * NS
CHUNKS = 80
K = 128
EPW = CHUNKS * K
EP = EPW * NW
PAD_ROW = NP - 1
DW = 128
ZR = NP // NS

PRW = NP // NW
PC, PK = 5, 64
SEG = 384
PAD_SEG = SEG - 1

BR = 512
GRID = NP // BR
EPS = 1e-5


def _sc_mesh():
    return plsc.VectorSubcoreMesh(
        core_axis_name="c", subcore_axis_name="s",
        num_cores=NC, num_subcores=NS)


@functools.cache
def _deg_kernel():
    @functools.partial(
        pl.kernel,
        out_type=jax.ShapeDtypeStruct((NC, NP, DW), F32),
        mesh=_sc_mesh(),
        scratch_types=[
            pltpu.VMEM((K,), I32),
            pltpu.VMEM((16, DW), F32),
            pltpu.VMEM_SHARED((NP, DW), F32),
            pltpu.SemaphoreType.DMA,
        ],
    )
    def deg(dsts_hbm, zeros_hbm, ones_hbm, out_hbm, idx_d, ones_v, acc, sem):
        c = lax.axis_index("c")
        s = lax.axis_index("s")
        pltpu.sync_copy(zeros_hbm, acc.at[pl.ds(s * ZR, ZR)])
        pltpu.sync_copy(ones_hbm, ones_v)
        plsc.subcore_barrier()

        def chunk(j, carry):
            pltpu.sync_copy(dsts_hbm.at[c, s, j], idx_d)
            ivs = [idx_d[pl.ds(t * 16, 16)] for t in range(K // 16)]
            for t in range(K // 16):
                pltpu.async_copy(ones_v, acc.at[ivs[t]], sem, add=True)
            for t in range(K // 16):
                pltpu.make_async_copy(ones_v, acc.at[ivs[t]], sem).wait()
            return carry

        lax.fori_loop(0, CHUNKS, chunk, 0)
        plsc.subcore_barrier()
        pltpu.sync_copy(acc.at[pl.ds(s * ZR, ZR)],
                        out_hbm.at[c, pl.ds(s * ZR, ZR)])

    return deg


@functools.cache
def _agg_kernel():
    @functools.partial(
        pl.kernel,
        out_type=jax.ShapeDtypeStruct((NC, NP, D), F32),
        mesh=_sc_mesh(),
        scratch_types=[
            pltpu.VMEM((K,), I32),
            pltpu.VMEM((K,), I32),
            pltpu.VMEM((K,), I32),
            pltpu.VMEM((K,), I32),
            pltpu.VMEM((K, D), F32),
            pltpu.VMEM((K, D), F32),
            pltpu.VMEM_SHARED((NP, D), F32),
            pltpu.SemaphoreType.DMA,
            pltpu.SemaphoreType.DMA,
            pltpu.SemaphoreType.DMA,
            pltpu.SemaphoreType.DMA,
        ],
    )
    def agg(g_hbm, srcs_hbm, dsts_hbm, zeros_hbm, out_hbm,
            idx_s0, idx_d0, idx_s1, idx_d1, rows0, rows1, acc,
            sem_g0, sem_g1, sem_s0, sem_s1):
        c = lax.axis_index("c")
        s = lax.axis_index("s")
        pltpu.sync_copy(zeros_hbm, acc.at[pl.ds(s * ZR, ZR)])
        plsc.subcore_barrier()

        pltpu.sync_copy(srcs_hbm.at[c, s, 0], idx_s0)
        pltpu.sync_copy(dsts_hbm.at[c, s, 0], idx_d0)
        pltpu.async_copy(g_hbm.at[idx_s0], rows0, sem_g0)
        pltpu.sync_copy(srcs_hbm.at[c, s, 1], idx_s1)
        pltpu.sync_copy(dsts_hbm.at[c, s, 1], idx_d1)
        pltpu.async_copy(g_hbm.at[idx_s1], rows1, sem_g1)

        def _scatter_granules(rows, idx_d, sem_s):
            ivs = [idx_d[pl.ds(t * 16, 16)] for t in range(K // 16)]
            for t in range(K // 16):
                pltpu.async_copy(rows.at[pl.ds(t * 16, 16)], acc.at[ivs[t]],
                                 sem_s, add=True)
            return ivs

        def _drain_granules(rows, ivs, sem_s):
            for t in range(K // 16):
                pltpu.make_async_copy(rows.at[pl.ds(t * 16, 16)],
                                      acc.at[ivs[t]], sem_s).wait()

        def pair(i, carry):
            pltpu.make_async_copy(g_hbm.at[idx_s0], rows0, sem_g0).wait()
            ivs0 = _scatter_granules(rows0, idx_d0, sem_s0)
            pltpu.make_async_copy(g_hbm.at[idx_s1], rows1, sem_g1).wait()
            ivs1 = _scatter_granules(rows1, idx_d1, sem_s1)

            @pl.when(i < CHUNKS // 2 - 1)
            def _():
                pltpu.sync_copy(srcs_hbm.at[c, s, 2 * i + 2], idx_s0)
                _drain_granules(rows0, ivs0, sem_s0)
                pltpu.sync_copy(dsts_hbm.at[c, s, 2 * i + 2], idx_d0)
                pltpu.async_copy(g_hbm.at[idx_s0], rows0, sem_g0)
                pltpu.sync_copy(srcs_hbm.at[c, s, 2 * i + 3], idx_s1)
                _drain_granules(rows1, ivs1, sem_s1)
                pltpu.sync_copy(dsts_hbm.at[c, s, 2 * i + 3], idx_d1)
                pltpu.async_copy(g_hbm.at[idx_s1], rows1, sem_g1)

            return carry

        lax.fori_loop(0, CHUNKS // 2, pair, 0)
        _drain_granules(rows0, [idx_d0[pl.ds(t * 16, 16)]
                                for t in range(K // 16)], sem_s0)
        _drain_granules(rows1, [idx_d1[pl.ds(t * 16, 16)]
                                for t in range(K // 16)], sem_s1)
        plsc.subcore_barrier()
        pltpu.sync_copy(acc.at[pl.ds(s * ZR, ZR)],
                        out_hbm.at[c, pl.ds(s * ZR, ZR)])

    return agg


@functools.cache
def _pool_kernel():
    @functools.partial(
        pl.kernel,
        out_type=(jax.ShapeDtypeStruct((NC, SEG, H), F32),
                  jax.ShapeDtypeStruct((NC, SEG, H), F32),
                  jax.ShapeDtypeStruct((NC, SEG, DW), F32)),
        mesh=_sc_mesh(),
        scratch_types=[
            pltpu.VMEM((PK,), I32),
            pltpu.VMEM((PK, H), F32),
            pltpu.VMEM((PK, H), F32),
            pltpu.VMEM((16, DW), F32),
            pltpu.VMEM_SHARED((SEG, H), F32),
            pltpu.VMEM_SHARED((SEG, H), F32),
            pltpu.VMEM_SHARED((SEG, DW), F32),
        ],
    )
    def pool(h2a_hbm, h2b_hbm, batch_hbm, zseg_hbm, zcnt_hbm, ones_hbm,
             sega_hbm, segb_hbm, cnt_hbm, idx_b, rows_a, rows_b, ones_v,
             acc_a, acc_b, acc_cnt):
        c = lax.axis_index("c")
        s = lax.axis_index("s")
        zr = SEG // NS
        pltpu.sync_copy(zseg_hbm, acc_a.at[pl.ds(s * zr, zr)])
        pltpu.sync_copy(zseg_hbm, acc_b.at[pl.ds(s * zr, zr)])
        pltpu.sync_copy(zcnt_hbm, acc_cnt.at[pl.ds(s * zr, zr)])
        pltpu.sync_copy(ones_hbm, ones_v)
        plsc.subcore_barrier()
        base = (c * NS + s) * PRW

        def chunk(j, carry):
            pltpu.sync_copy(batch_hbm.at[c, s, j], idx_b)
            pltpu.sync_copy(h2a_hbm.at[pl.ds(base + j * PK, PK)], rows_a)
            pltpu.sync_copy(h2b_hbm.at[pl.ds(base + j * PK, PK)], rows_b)
            for t in range(PK // 16):
                iv = idx_b[pl.ds(t * 16, 16)]
                pltpu.sync_copy(rows_a.at[pl.ds(t * 16, 16)], acc_a.at[iv],
                                add=True)
                pltpu.sync_copy(rows_b.at[pl.ds(t * 16, 16)], acc_b.at[iv],
                                add=True)
                pltpu.sync_copy(ones_v, acc_cnt.at[iv], add=True)
            return carry

        lax.fori_loop(0, PC, chunk, 0)
        plsc.subcore_barrier()
        pltpu.sync_copy(acc_a.at[pl.ds(s * zr, zr)],
                        sega_hbm.at[c, pl.ds(s * zr, zr)])
        pltpu.sync_copy(acc_b.at[pl.ds(s * zr, zr)],
                        segb_hbm.at[c, pl.ds(s * zr, zr)])
        pltpu.sync_copy(acc_cnt.at[pl.ds(s * zr, zr)],
                        cnt_hbm.at[c, pl.ds(s * zr, zr)])

    return pool


def _row_spec(w):
    return pl.BlockSpec((BR, w), lambda i: (i, 0))


def _whole(shape):
    return pl.BlockSpec(shape, lambda i: tuple(0 for _ in shape))


def _prep_body(x_ref, w1_ref, d0_ref, d1_ref, g1_ref, dinv_ref):
    d = d0_ref[:, 0:1] + d1_ref[:, 0:1] + 1.0
    dinv = jnp.broadcast_to(lax.rsqrt(d), (BR, D))
    xw = jnp.dot(x_ref[...], w1_ref[...], preferred_element_type=F32)
    g1_ref[...] = xw * dinv
    dinv_ref[...] = dinv


def _tc_prep(x_p, w1, deg0, deg1):
    return pl.pallas_call(
        _prep_body,
        grid=(GRID,),
        in_specs=[_row_spec(D), _whole((D, H)), _row_spec(DW), _row_spec(DW)],
        out_specs=[_row_spec(H), _row_spec(D)],
        out_shape=[jax.ShapeDtypeStruct((NP, H), F32),
                   jax.ShapeDtypeStruct((NP, D), F32)],
    )(x_p, w1, deg0, deg1)


def _epi1_body(s0_ref, s1_ref, g1_ref, dinv_ref, b1_ref,
               r1_ref, sum_ref, sq_ref):
    i = pl.program_id(0)
    r = jnp.maximum(
        dinv_ref[...] * (s0_ref[...] + s1_ref[...] + g1_ref[...])
        + b1_ref[...], 0.0)
    r1_ref[...] = r
    rowid = lax.broadcasted_iota(I32, (BR, H), 0) + i * BR
    rm = jnp.where(rowid < N, r, 0.0)

    @pl.when(i == 0)
    def _():
        sum_ref[...] = jnp.zeros((8, H), F32)
        sq_ref[...] = jnp.zeros((8, H), F32)

    sum_ref[0:1, :] += jnp.sum(rm, axis=0, keepdims=True)
    sq_ref[0:1, :] += jnp.sum(rm * rm, axis=0, keepdims=True)


def _tc_epi1(s1_parts, g1, dinv_col, b1):
    return pl.pallas_call(
        _epi1_body,
        grid=(GRID,),
        in_specs=[_row_spec(H), _row_spec(H), _row_spec(H), _row_spec(D),
                  _whole((1, H))],
        out_specs=[_row_spec(H), _whole((8, H)), _whole((8, H))],
        out_shape=[jax.ShapeDtypeStruct((NP, H), F32),
                   jax.ShapeDtypeStruct((8, H), F32),
                   jax.ShapeDtypeStruct((8, H), F32)],
    )(s1_parts[0], s1_parts[1], g1, dinv_col, b1)


def _bnmm_body(r1_ref, sum_ref, sq_ref, ga_ref, be_ref, w2a_ref, w2b_ref,
               dinv_ref, g2a_ref, g2b_ref):
    mu = sum_ref[0:1, :] * (1.0 / N)
    var = sq_ref[0:1, :] * (1.0 / N) - mu * mu
    sfac = ga_ref[...] * lax.rsqrt(var + EPS)
    t = be_ref[...] - mu * sfac
    h = r1_ref[...] * sfac + t
    dv = dinv_ref[...]
    g2a_ref[...] = jnp.dot(h, w2a_ref[...], preferred_element_type=F32) * dv
    g2b_ref[...] = jnp.dot(h, w2b_ref[...], preferred_element_type=F32) * dv


def _tc_bnmm(r1, ssum, ssq, gamma1, beta1, w2a, w2b, dinv_col):
    return pl.pallas_call(
        _bnmm_body,
        grid=(GRID,),
        in_specs=[_row_spec(H), _whole((8, H)), _whole((8, H)),
                  _whole((1, H)), _whole((1, H)),
                  _whole((H, H)), _whole((H, H)), _row_spec(D)],
        out_specs=[_row_spec(H), _row_spec(H)],
        out_shape=[jax.ShapeDtypeStruct((NP, H), F32),
                   jax.ShapeDtypeStruct((NP, H), F32)],
    )(r1, ssum, ssq, gamma1, beta1, w2a, w2b, dinv_col)


def _epi2_body(sa0_ref, sa1_ref, sb0_ref, sb1_ref, g2a_ref, g2b_ref,
               dinv_ref, b2a_ref, b2b_ref, r2_ref, sum_ref, sq_ref):
    i = pl.program_id(0)
    dv = dinv_ref[...]
    ra = jnp.maximum(
        dv * (sa0_ref[...] + sa1_ref[...] + g2a_ref[...]) + b2a_ref[...], 0.0)
    rb = jnp.maximum(
        dv * (sb0_ref[...] + sb1_ref[...] + g2b_ref[...]) + b2b_ref[...], 0.0)
    r2_ref[:, 0:H] = ra
    r2_ref[:, H:2 * H] = rb
    rowid = lax.broadcasted_iota(I32, (BR, H), 0) + i * BR
    ram = jnp.where(rowid < N, ra, 0.0)
    rbm = jnp.where(rowid < N, rb, 0.0)

    @pl.when(i == 0)
    def _():
        sum_ref[...] = jnp.zeros((8, 2 * H), F32)
        sq_ref[...] = jnp.zeros((8, 2 * H), F32)

    sum_ref[0:1, 0:H] += jnp.sum(ram, axis=0, keepdims=True)
    sum_ref[0:1, H:2 * H] += jnp.sum(rbm, axis=0, keepdims=True)
    sq_ref[0:1, 0:H] += jnp.sum(ram * ram, axis=0, keepdims=True)
    sq_ref[0:1, H:2 * H] += jnp.sum(rbm * rbm, axis=0, keepdims=True)


def _tc_epi2(s2a, s2b, g2a, g2b, dinv_col, b2a, b2b):
    return pl.pallas_call(
        _epi2_body,
        grid=(GRID,),
        in_specs=[_row_spec(H), _row_spec(H), _row_spec(H), _row_spec(H),
                  _row_spec(H), _row_spec(H), _row_spec(D),
                  _whole((1, H)), _whole((1, H))],
        out_specs=[_row_spec(2 * H), _whole((8, 2 * H)), _whole((8, 2 * H))],
        out_shape=[jax.ShapeDtypeStruct((NP, 2 * H), F32),
                   jax.ShapeDtypeStruct((8, 2 * H), F32),
                   jax.ShapeDtypeStruct((8, 2 * H), F32)],
    )(s2a[0], s2a[1], s2b[0], s2b[1], g2a, g2b, dinv_col, b2a, b2b)


def _bn2_body(r2_ref, sum_ref, sq_ref, ga_ref, be_ref, h2a_ref, h2b_ref):
    mu = sum_ref[0:1, :] * (1.0 / N)
    var = sq_ref[0:1, :] * (1.0 / N) - mu * mu
    sfac = ga_ref[...] * lax.rsqrt(var + EPS)
    t = be_ref[...] - mu * sfac
    h2 = r2_ref[...] * sfac + t
    h2a_ref[...] = h2[:, 0:H]
    h2b_ref[...] = h2[:, H:2 * H]


def _tc_bn2(r2, ssum, ssq, gamma2, beta2):
    return pl.pallas_call(
        _bn2_body,
        grid=(GRID,),
        in_specs=[_row_spec(2 * H), _whole((8, 2 * H)), _whole((8, 2 * H)),
                  _whole((1, 2 * H)), _whole((1, 2 * H))],
        out_specs=[_row_spec(H), _row_spec(H)],
        out_shape=[jax.ShapeDtypeStruct((NP, H), F32),
                   jax.ShapeDtypeStruct((NP, H), F32)],
    )(r2, ssum, ssq, gamma2, beta2)


def _bn_rows(z, gamma, beta):
    mu = jnp.mean(z, axis=0, keepdims=True)
    var = jnp.mean(z * z, axis=0, keepdims=True) - mu * mu
    return (z - mu) * lax.rsqrt(var + EPS) * gamma + beta


def _head_body(s0a_ref, s0b_ref, s1a_ref, s1b_ref, cnt0_ref, cnt1_ref,
               rdk_ref, w3ea_ref, w3eb_ref, w3r_ref, b3_ref, g3_ref, be3_ref,
               w4_ref, b4_ref, g4_ref, be4_ref, w5_ref, b5_ref, out_ref):
    cnt = cnt0_ref[:, 0:1] + cnt1_ref[:, 0:1]
    inv = 1.0 / jnp.maximum(cnt, 1.0)
    emb_a = (s0a_ref[...] + s1a_ref[...]) * inv
    emb_b = (s0b_ref[...] + s1b_ref[...]) * inv
    z = jnp.dot(emb_a, w3ea_ref[...], preferred_element_type=F32)
    z += jnp.dot(emb_b, w3eb_ref[...], preferred_element_type=F32)
    z += jnp.dot(rdk_ref[...], w3r_ref[...], preferred_element_type=F32)
    z = jnp.maximum(z + b3_ref[...], 0.0)
    z = _bn_rows(z, g3_ref[...], be3_ref[...])
    z = jnp.maximum(
        jnp.dot(z, w4_ref[...], preferred_element_type=F32) + b4_ref[...], 0.0)
    z = _bn_rows(z, g4_ref[...], be4_ref[...])
    out_ref[...] = (jnp.dot(z, w5_ref[...], preferred_element_type=F32)
                    + b5_ref[...])


def _tc_head(s0a, s0b, s1a, s1b, cnt0, cnt1, rdk_p, w3ea, w3eb, w3r,
             b3, g3, be3, w4, b4, g4, be4, w5_p, b5_p):
    shapes = [(B, H), (B, H), (B, H), (B, H), (B, DW), (B, DW), (B, 2 * H),
              (H, 2 * H), (H, 2 * H), (2 * H, 2 * H), (1, 2 * H), (1, 2 * H),
              (1, 2 * H), (2 * H, H), (1, H), (1, H), (1, H),
              (H, H), (1, H)]
    return pl.pallas_call(
        _head_body,
        grid=(1,),
        in_specs=[_whole(s) for s in shapes],
        out_specs=_whole((B, H)),
        out_shape=jax.ShapeDtypeStruct((B, H), F32),
    )(s0a, s0b, s1a, s1b, cnt0, cnt1, rdk_p, w3ea, w3eb, w3r,
      b3, g3, be3, w4, b4, g4, be4, w5_p, b5_p)


def kernel(x, edge_index, batch, rdkit_feats, W1, b1, gamma1, beta1,
           W2, b2, gamma2, beta2, W3, b3, gamma3, beta3,
           W4, b4, gamma4, beta4, W5, b5):
    ei = edge_index.astype(I32)
    pad_e = N + (jnp.arange(EP - E, dtype=I32) % (NP - N))
    srcs = jnp.concatenate([ei[0], pad_e]).reshape(NC, NS, CHUNKS, K)
    dsts = jnp.concatenate([ei[1], pad_e]).reshape(NC, NS, CHUNKS, K)
    x_p = jnp.pad(x, ((0, NP - N), (0, 0)))
    batch_p = jnp.concatenate(
        [batch.astype(I32),
         B + (jnp.arange(NP - N, dtype=I32) % (SEG - B))]
    ).reshape(NC, NS, PC, PK)
    rdk_p = jnp.pad(rdkit_feats, ((0, 0), (0, 2 * H - R)))

    zeros_deg = jnp.zeros((ZR, DW), F32)
    ones_deg = jnp.ones((16, DW), F32)
    zeros_agg = jnp.zeros((ZR, D), F32)
    zseg = jnp.zeros((SEG // NS, H), F32)
    zcnt = jnp.zeros((SEG // NS, DW), F32)
    ones_pool = jnp.ones((16, DW), F32)

    b1r = b1.reshape(1, H)
    g1r, be1r = gamma1.reshape(1, H), beta1.reshape(1, H)
    w2a, w2b = W2[:, :H], W2[:, H:]
    b2a, b2b = b2[:H].reshape(1, H), b2[H:].reshape(1, H)
    g2r, be2r = gamma2.reshape(1, 2 * H), beta2.reshape(1, 2 * H)
    w3ea, w3eb = W3[:H], W3[H:2 * H]
    w3r = jnp.pad(W3[2 * H:], ((0, 2 * H - R), (0, 0)))
    b3r = b3.reshape(1, 2 * H)
    g3r, be3r = gamma3.reshape(1, 2 * H), beta3.reshape(1, 2 * H)
    b4r = b4.reshape(1, H)
    g4r, be4r = gamma4.reshape(1, H), beta4.reshape(1, H)
    w5_p = jnp.pad(W5, ((0, 0), (0, H - 1)))
    b5_p = jnp.pad(b5, (0, H - 1)).reshape(1, H)

    deg = _deg_kernel()(dsts, zeros_deg, ones_deg)
    g1, dinv_col = _tc_prep(x_p, W1, deg[0], deg[1])

    s1 = _agg_kernel()(g1, srcs, dsts, zeros_agg)
    r1, ssum1, ssq1 = _tc_epi1((s1[0], s1[1]), g1, dinv_col, b1r)

    g2a, g2b = _tc_bnmm(r1, ssum1, ssq1, g1r, be1r, w2a, w2b, dinv_col)

    s2a = _agg_kernel()(g2a, srcs, dsts, zeros_agg)
    s2b = _agg_kernel()(g2b, srcs, dsts, zeros_agg)
    r2, ssum2, ssq2 = _tc_epi2((s2a[0], s2a[1]), (s2b[0], s2b[1]),
                               g2a, g2b, dinv_col, b2a, b2b)
    h2a, h2b = _tc_bn2(r2, ssum2, ssq2, g2r, be2r)

    seg_a, seg_b, cnt = _pool_kernel()(h2a, h2b, batch_p, zseg, zcnt,
                                       ones_pool)

    out = _tc_head(seg_a[0, :B], seg_b[0, :B], seg_a[1, :B], seg_b[1, :B],
                   cnt[0, :B], cnt[1, :B], rdk_p, w3ea, w3eb, w3r,
                   b3r, g3r, be3r, W4, b4r, g4r, be4r, w5_p, b5_p)
    return out[:, 0]


# speedup vs baseline: 16.1280x; 1.0991x over previous
import functools

import jax
import jax.numpy as jnp
from jax import lax
from jax.experimental import pallas as pl
from jax.experimental.pallas import tpu as pltpu
from jax.experimental.pallas import tpu_sc as plsc

F32 = jnp.float32
I32 = jnp.int32

N = 10000
E = 320000
D = 128
B = 256
R = 200
H = 128

NP = 10240
NC, NS = 2, 16
NW = NC * NS
CHUNKS = 80
K = 128
EPW = CHUNKS * K
EP = EPW * NW
PAD_ROW = NP - 1
DW = 128
ZR = NP // NS

PRW = NP // NW
PC, PK = 5, 64
SEG = 384
PAD_SEG = SEG - 1

BR = 512
GRID = NP // BR
EPS = 1e-5


def _sc_mesh():
    return plsc.VectorSubcoreMesh(
        core_axis_name="c", subcore_axis_name="s",
        num_cores=NC, num_subcores=NS)


@functools.cache
def _deg_kernel():
    @functools.partial(
        pl.kernel,
        out_type=jax.ShapeDtypeStruct((NC, NP, DW), F32),
        mesh=_sc_mesh(),
        scratch_types=[
            pltpu.VMEM((CHUNKS, K), I32),
            pltpu.VMEM((16, DW), F32),
            pltpu.VMEM_SHARED((NP, DW), F32),
            pltpu.SemaphoreType.DMA,
        ],
    )
    def deg(dsts_hbm, zeros_hbm, ones_hbm, out_hbm, idx_d, ones_v, acc, sem):
        c = lax.axis_index("c")
        s = lax.axis_index("s")
        pltpu.sync_copy(zeros_hbm, acc.at[pl.ds(s * ZR, ZR)])
        pltpu.sync_copy(dsts_hbm.at[c, s], idx_d)
        pltpu.sync_copy(ones_hbm, ones_v)
        plsc.subcore_barrier()

        def chunk(j, carry):
            ivs = [idx_d[j, pl.ds(t * 16, 16)] for t in range(K // 16)]
            for t in range(K // 16):
                pltpu.async_copy(ones_v, acc.at[ivs[t]], sem, add=True)
            for t in range(K // 16):
                pltpu.make_async_copy(ones_v, acc.at[ivs[t]], sem).wait()
            return carry

        lax.fori_loop(0, CHUNKS, chunk, 0)
        plsc.subcore_barrier()
        pltpu.sync_copy(acc.at[pl.ds(s * ZR, ZR)],
                        out_hbm.at[c, pl.ds(s * ZR, ZR)])

    return deg


@functools.cache
def _agg_kernel():
    @functools.partial(
        pl.kernel,
        out_type=jax.ShapeDtypeStruct((NC, NP, D), F32),
        mesh=_sc_mesh(),
        scratch_types=[
            pltpu.VMEM((K,), I32),
            pltpu.VMEM((K,), I32),
            pltpu.VMEM((K,), I32),
            pltpu.VMEM((K,), I32),
            pltpu.VMEM((K, D), F32),
            pltpu.VMEM((K, D), F32),
            pltpu.VMEM_SHARED((NP, D), F32),
            pltpu.SemaphoreType.DMA,
            pltpu.SemaphoreType.DMA,
            pltpu.SemaphoreType.DMA,
            pltpu.SemaphoreType.DMA,
            pltpu.SemaphoreType.DMA,
            pltpu.SemaphoreType.DMA,
        ],
    )
    def agg(g_hbm, srcs_hbm, dsts_hbm, zeros_hbm, out_hbm,
            idx_s0, idx_d0, idx_s1, idx_d1, rows0, rows1, acc,
            sem_g0, sem_g1, sem_s0, sem_s1, sem_i0, sem_i1):
        c = lax.axis_index("c")
        s = lax.axis_index("s")
        pltpu.sync_copy(zeros_hbm, acc.at[pl.ds(s * ZR, ZR)])
        plsc.subcore_barrier()

        pltpu.sync_copy(srcs_hbm.at[c, s, 0], idx_s0)
        pltpu.sync_copy(dsts_hbm.at[c, s, 0], idx_d0)
        pltpu.async_copy(g_hbm.at[idx_s0], rows0, sem_g0)
        pltpu.sync_copy(srcs_hbm.at[c, s, 1], idx_s1)
        pltpu.sync_copy(dsts_hbm.at[c, s, 1], idx_d1)
        pltpu.async_copy(g_hbm.at[idx_s1], rows1, sem_g1)

        def _scatter_granules(rows, idx_d, sem_s):
            ivs = [idx_d[pl.ds(t * 16, 16)] for t in range(K // 16)]
            for t in range(K // 16):
                pltpu.async_copy(rows.at[pl.ds(t * 16, 16)], acc.at[ivs[t]],
                                 sem_s, add=True)
            return ivs

        def _drain_granules(rows, ivs, sem_s):
            for t in range(K // 16):
                pltpu.make_async_copy(rows.at[pl.ds(t * 16, 16)],
                                      acc.at[ivs[t]], sem_s).wait()

        def pair(i, carry):
            last = i >= CHUNKS // 2 - 1
            pltpu.make_async_copy(g_hbm.at[idx_s0], rows0, sem_g0).wait()
            ivs0 = _scatter_granules(rows0, idx_d0, sem_s0)

            @pl.when(jnp.logical_not(last))
            def _():
                pltpu.async_copy(srcs_hbm.at[c, s, 2 * i + 2], idx_s0,
                                 sem_i0)
                pltpu.async_copy(dsts_hbm.at[c, s, 2 * i + 2], idx_d0,
                                 sem_i0)

            pltpu.make_async_copy(g_hbm.at[idx_s1], rows1, sem_g1).wait()
            ivs1 = _scatter_granules(rows1, idx_d1, sem_s1)

            @pl.when(jnp.logical_not(last))
            def _():
                pltpu.async_copy(srcs_hbm.at[c, s, 2 * i + 3], idx_s1,
                                 sem_i1)
                pltpu.async_copy(dsts_hbm.at[c, s, 2 * i + 3], idx_d1,
                                 sem_i1)
                _drain_granules(rows0, ivs0, sem_s0)
                pltpu.make_async_copy(srcs_hbm.at[c, s, 0], idx_s0,
                                      sem_i0).wait()
                pltpu.make_async_copy(dsts_hbm.at[c, s, 0], idx_d0,
                                      sem_i0).wait()
                pltpu.async_copy(g_hbm.at[idx_s0], rows0, sem_g0)
                _drain_granules(rows1, ivs1, sem_s1)
                pltpu.make_async_copy(srcs_hbm.at[c, s, 0], idx_s1,
                                      sem_i1).wait()
                pltpu.make_async_copy(dsts_hbm.at[c, s, 0], idx_d1,
                                      sem_i1).wait()
                pltpu.async_copy(g_hbm.at[idx_s1], rows1, sem_g1)

            return carry

        lax.fori_loop(0, CHUNKS // 2, pair, 0)
        _drain_granules(rows0, [idx_d0[pl.ds(t * 16, 16)]
                                for t in range(K // 16)], sem_s0)
        _drain_granules(rows1, [idx_d1[pl.ds(t * 16, 16)]
                                for t in range(K // 16)], sem_s1)
        plsc.subcore_barrier()
        pltpu.sync_copy(acc.at[pl.ds(s * ZR, ZR)],
                        out_hbm.at[c, pl.ds(s * ZR, ZR)])

    return agg


@functools.cache
def _pool_kernel():
    @functools.partial(
        pl.kernel,
        out_type=(jax.ShapeDtypeStruct((NC, SEG, H), F32),
                  jax.ShapeDtypeStruct((NC, SEG, H), F32),
                  jax.ShapeDtypeStruct((NC, SEG, DW), F32)),
        mesh=_sc_mesh(),
        scratch_types=[
            pltpu.VMEM((PK,), I32),
            pltpu.VMEM((PK, H), F32),
            pltpu.VMEM((PK, H), F32),
            pltpu.VMEM((16, DW), F32),
            pltpu.VMEM_SHARED((SEG, H), F32),
            pltpu.VMEM_SHARED((SEG, H), F32),
            pltpu.VMEM_SHARED((SEG, DW), F32),
        ],
    )
    def pool(h2a_hbm, h2b_hbm, batch_hbm, zseg_hbm, zcnt_hbm, ones_hbm,
             sega_hbm, segb_hbm, cnt_hbm, idx_b, rows_a, rows_b, ones_v,
             acc_a, acc_b, acc_cnt):
        c = lax.axis_index("c")
        s = lax.axis_index("s")
        zr = SEG // NS
        pltpu.sync_copy(zseg_hbm, acc_a.at[pl.ds(s * zr, zr)])
        pltpu.sync_copy(zseg_hbm, acc_b.at[pl.ds(s * zr, zr)])
        pltpu.sync_copy(zcnt_hbm, acc_cnt.at[pl.ds(s * zr, zr)])
        pltpu.sync_copy(ones_hbm, ones_v)
        plsc.subcore_barrier()
        base = (c * NS + s) * PRW

        def chunk(j, carry):
            pltpu.sync_copy(batch_hbm.at[c, s, j], idx_b)
            pltpu.sync_copy(h2a_hbm.at[pl.ds(base + j * PK, PK)], rows_a)
            pltpu.sync_copy(h2b_hbm.at[pl.ds(base + j * PK, PK)], rows_b)
            for t in range(PK // 16):
                iv = idx_b[pl.ds(t * 16, 16)]
                pltpu.sync_copy(rows_a.at[pl.ds(t * 16, 16)], acc_a.at[iv],
                                add=True)
                pltpu.sync_copy(rows_b.at[pl.ds(t * 16, 16)], acc_b.at[iv],
                                add=True)
                pltpu.sync_copy(ones_v, acc_cnt.at[iv], add=True)
            return carry

        lax.fori_loop(0, PC, chunk, 0)
        plsc.subcore_barrier()
        pltpu.sync_copy(acc_a.at[pl.ds(s * zr, zr)],
                        sega_hbm.at[c, pl.ds(s * zr, zr)])
        pltpu.sync_copy(acc_b.at[pl.ds(s * zr, zr)],
                        segb_hbm.at[c, pl.ds(s * zr, zr)])
        pltpu.sync_copy(acc_cnt.at[pl.ds(s * zr, zr)],
                        cnt_hbm.at[c, pl.ds(s * zr, zr)])

    return pool


def _row_spec(w):
    return pl.BlockSpec((BR, w), lambda i: (i, 0))


def _whole(shape):
    return pl.BlockSpec(shape, lambda i: tuple(0 for _ in shape))


def _prep_body(x_ref, w1_ref, d0_ref, d1_ref, g1_ref, dinv_ref):
    d = d0_ref[:, 0:1] + d1_ref[:, 0:1] + 1.0
    dinv = jnp.broadcast_to(lax.rsqrt(d), (BR, D))
    xw = jnp.dot(x_ref[...], w1_ref[...], preferred_element_type=F32)
    g1_ref[...] = xw * dinv
    dinv_ref[...] = dinv


def _tc_prep(x_p, w1, deg0, deg1):
    return pl.pallas_call(
        _prep_body,
        grid=(GRID,),
        in_specs=[_row_spec(D), _whole((D, H)), _row_spec(DW), _row_spec(DW)],
        out_specs=[_row_spec(H), _row_spec(D)],
        out_shape=[jax.ShapeDtypeStruct((NP, H), F32),
                   jax.ShapeDtypeStruct((NP, D), F32)],
    )(x_p, w1, deg0, deg1)


def _epi1_body(s0_ref, s1_ref, g1_ref, dinv_ref, b1_ref,
               r1_ref, sum_ref, sq_ref):
    i = pl.program_id(0)
    r = jnp.maximum(
        dinv_ref[...] * (s0_ref[...] + s1_ref[...] + g1_ref[...])
        + b1_ref[...], 0.0)
    r1_ref[...] = r
    rowid = lax.broadcasted_iota(I32, (BR, H), 0) + i * BR
    rm = jnp.where(rowid < N, r, 0.0)

    @pl.when(i == 0)
    def _():
        sum_ref[...] = jnp.zeros((8, H), F32)
        sq_ref[...] = jnp.zeros((8, H), F32)

    sum_ref[0:1, :] += jnp.sum(rm, axis=0, keepdims=True)
    sq_ref[0:1, :] += jnp.sum(rm * rm, axis=0, keepdims=True)


def _tc_epi1(s1_parts, g1, dinv_col, b1):
    return pl.pallas_call(
        _epi1_body,
        grid=(GRID,),
        in_specs=[_row_spec(H), _row_spec(H), _row_spec(H), _row_spec(D),
                  _whole((1, H))],
        out_specs=[_row_spec(H), _whole((8, H)), _whole((8, H))],
        out_shape=[jax.ShapeDtypeStruct((NP, H), F32),
                   jax.ShapeDtypeStruct((8, H), F32),
                   jax.ShapeDtypeStruct((8, H), F32)],
    )(s1_parts[0], s1_parts[1], g1, dinv_col, b1)


def _bnmm_body(r1_ref, sum_ref, sq_ref, ga_ref, be_ref, w2a_ref, w2b_ref,
               dinv_ref, g2a_ref, g2b_ref):
    mu = sum_ref[0:1, :] * (1.0 / N)
    var = sq_ref[0:1, :] * (1.0 / N) - mu * mu
    sfac = ga_ref[...] * lax.rsqrt(var + EPS)
    t = be_ref[...] - mu * sfac
    h = r1_ref[...] * sfac + t
    dv = dinv_ref[...]
    g2a_ref[...] = jnp.dot(h, w2a_ref[...], preferred_element_type=F32) * dv
    g2b_ref[...] = jnp.dot(h, w2b_ref[...], preferred_element_type=F32) * dv


def _tc_bnmm(r1, ssum, ssq, gamma1, beta1, w2a, w2b, dinv_col):
    return pl.pallas_call(
        _bnmm_body,
        grid=(GRID,),
        in_specs=[_row_spec(H), _whole((8, H)), _whole((8, H)),
                  _whole((1, H)), _whole((1, H)),
                  _whole((H, H)), _whole((H, H)), _row_spec(D)],
        out_specs=[_row_spec(H), _row_spec(H)],
        out_shape=[jax.ShapeDtypeStruct((NP, H), F32),
                   jax.ShapeDtypeStruct((NP, H), F32)],
    )(r1, ssum, ssq, gamma1, beta1, w2a, w2b, dinv_col)


def _epi2_body(sa0_ref, sa1_ref, sb0_ref, sb1_ref, g2a_ref, g2b_ref,
               dinv_ref, b2a_ref, b2b_ref, r2_ref, sum_ref, sq_ref):
    i = pl.program_id(0)
    dv = dinv_ref[...]
    ra = jnp.maximum(
        dv * (sa0_ref[...] + sa1_ref[...] + g2a_ref[...]) + b2a_ref[...], 0.0)
    rb = jnp.maximum(
        dv * (sb0_ref[...] + sb1_ref[...] + g2b_ref[...]) + b2b_ref[...], 0.0)
    r2_ref[:, 0:H] = ra
    r2_ref[:, H:2 * H] = rb
    rowid = lax.broadcasted_iota(I32, (BR, H), 0) + i * BR
    ram = jnp.where(rowid < N, ra, 0.0)
    rbm = jnp.where(rowid < N, rb, 0.0)

    @pl.when(i == 0)
    def _():
        sum_ref[...] = jnp.zeros((8, 2 * H), F32)
        sq_ref[...] = jnp.zeros((8, 2 * H), F32)

    sum_ref[0:1, 0:H] += jnp.sum(ram, axis=0, keepdims=True)
    sum_ref[0:1, H:2 * H] += jnp.sum(rbm, axis=0, keepdims=True)
    sq_ref[0:1, 0:H] += jnp.sum(ram * ram, axis=0, keepdims=True)
    sq_ref[0:1, H:2 * H] += jnp.sum(rbm * rbm, axis=0, keepdims=True)


def _tc_epi2(s2a, s2b, g2a, g2b, dinv_col, b2a, b2b):
    return pl.pallas_call(
        _epi2_body,
        grid=(GRID,),
        in_specs=[_row_spec(H), _row_spec(H), _row_spec(H), _row_spec(H),
                  _row_spec(H), _row_spec(H), _row_spec(D),
                  _whole((1, H)), _whole((1, H))],
        out_specs=[_row_spec(2 * H), _whole((8, 2 * H)), _whole((8, 2 * H))],
        out_shape=[jax.ShapeDtypeStruct((NP, 2 * H), F32),
                   jax.ShapeDtypeStruct((8, 2 * H), F32),
                   jax.ShapeDtypeStruct((8, 2 * H), F32)],
    )(s2a[0], s2a[1], s2b[0], s2b[1], g2a, g2b, dinv_col, b2a, b2b)


def _bn2_body(r2_ref, sum_ref, sq_ref, ga_ref, be_ref, h2a_ref, h2b_ref):
    mu = sum_ref[0:1, :] * (1.0 / N)
    var = sq_ref[0:1, :] * (1.0 / N) - mu * mu
    sfac = ga_ref[...] * lax.rsqrt(var + EPS)
    t = be_ref[...] - mu * sfac
    h2 = r2_ref[...] * sfac + t
    h2a_ref[...] = h2[:, 0:H]
    h2b_ref[...] = h2[:, H:2 * H]


def _tc_bn2(r2, ssum, ssq, gamma2, beta2):
    return pl.pallas_call(
        _bn2_body,
        grid=(GRID,),
        in_specs=[_row_spec(2 * H), _whole((8, 2 * H)), _whole((8, 2 * H)),
                  _whole((1, 2 * H)), _whole((1, 2 * H))],
        out_specs=[_row_spec(H), _row_spec(H)],
        out_shape=[jax.ShapeDtypeStruct((NP, H), F32),
                   jax.ShapeDtypeStruct((NP, H), F32)],
    )(r2, ssum, ssq, gamma2, beta2)


def _bn_rows(z, gamma, beta):
    mu = jnp.mean(z, axis=0, keepdims=True)
    var = jnp.mean(z * z, axis=0, keepdims=True) - mu * mu
    return (z - mu) * lax.rsqrt(var + EPS) * gamma + beta


def _head_body(s0a_ref, s0b_ref, s1a_ref, s1b_ref, cnt0_ref, cnt1_ref,
               rdk_ref, w3ea_ref, w3eb_ref, w3r_ref, b3_ref, g3_ref, be3_ref,
               w4_ref, b4_ref, g4_ref, be4_ref, w5_ref, b5_ref, out_ref):
    cnt = cnt0_ref[:, 0:1] + cnt1_ref[:, 0:1]
    inv = 1.0 / jnp.maximum(cnt, 1.0)
    emb_a = (s0a_ref[...] + s1a_ref[...]) * inv
    emb_b = (s0b_ref[...] + s1b_ref[...]) * inv
    z = jnp.dot(emb_a, w3ea_ref[...], preferred_element_type=F32)
    z += jnp.dot(emb_b, w3eb_ref[...], preferred_element_type=F32)
    z += jnp.dot(rdk_ref[...], w3r_ref[...], preferred_element_type=F32)
    z = jnp.maximum(z + b3_ref[...], 0.0)
    z = _bn_rows(z, g3_ref[...], be3_ref[...])
    z = jnp.maximum(
        jnp.dot(z, w4_ref[...], preferred_element_type=F32) + b4_ref[...], 0.0)
    z = _bn_rows(z, g4_ref[...], be4_ref[...])
    out_ref[...] = (jnp.dot(z, w5_ref[...], preferred_element_type=F32)
                    + b5_ref[...])


def _tc_head(s0a, s0b, s1a, s1b, cnt0, cnt1, rdk_p, w3ea, w3eb, w3r,
             b3, g3, be3, w4, b4, g4, be4, w5_p, b5_p):
    shapes = [(B, H), (B, H), (B, H), (B, H), (B, DW), (B, DW), (B, 2 * H),
              (H, 2 * H), (H, 2 * H), (2 * H, 2 * H), (1, 2 * H), (1, 2 * H),
              (1, 2 * H), (2 * H, H), (1, H), (1, H), (1, H),
              (H, H), (1, H)]
    return pl.pallas_call(
        _head_body,
        grid=(1,),
        in_specs=[_whole(s) for s in shapes],
        out_specs=_whole((B, H)),
        out_shape=jax.ShapeDtypeStruct((B, H), F32),
    )(s0a, s0b, s1a, s1b, cnt0, cnt1, rdk_p, w3ea, w3eb, w3r,
      b3, g3, be3, w4, b4, g4, be4, w5_p, b5_p)


def kernel(x, edge_index, batch, rdkit_feats, W1, b1, gamma1, beta1,
           W2, b2, gamma2, beta2, W3, b3, gamma3, beta3,
           W4, b4, gamma4, beta4, W5, b5):
    ei = edge_index.astype(I32)
    pad_e = N + (jnp.arange(EP - E, dtype=I32) % (NP - N))
    srcs = jnp.concatenate([ei[0], pad_e]).reshape(NC, NS, CHUNKS, K)
    dsts = jnp.concatenate([ei[1], pad_e]).reshape(NC, NS, CHUNKS, K)
    x_p = jnp.pad(x, ((0, NP - N), (0, 0)))
    batch_p = jnp.concatenate(
        [batch.astype(I32),
         B + (jnp.arange(NP - N, dtype=I32) % (SEG - B))]
    ).reshape(NC, NS, PC, PK)
    rdk_p = jnp.pad(rdkit_feats, ((0, 0), (0, 2 * H - R)))

    zeros_deg = jnp.zeros((ZR, DW), F32)
    ones_deg = jnp.ones((16, DW), F32)
    zeros_agg = jnp.zeros((ZR, D), F32)
    zseg = jnp.zeros((SEG // NS, H), F32)
    zcnt = jnp.zeros((SEG // NS, DW), F32)
    ones_pool = jnp.ones((16, DW), F32)

    b1r = b1.reshape(1, H)
    g1r, be1r = gamma1.reshape(1, H), beta1.reshape(1, H)
    w2a, w2b = W2[:, :H], W2[:, H:]
    b2a, b2b = b2[:H].reshape(1, H), b2[H:].reshape(1, H)
    g2r, be2r = gamma2.reshape(1, 2 * H), beta2.reshape(1, 2 * H)
    w3ea, w3eb = W3[:H], W3[H:2 * H]
    w3r = jnp.pad(W3[2 * H:], ((0, 2 * H - R), (0, 0)))
    b3r = b3.reshape(1, 2 * H)
    g3r, be3r = gamma3.reshape(1, 2 * H), beta3.reshape(1, 2 * H)
    b4r = b4.reshape(1, H)
    g4r, be4r = gamma4.reshape(1, H), beta4.reshape(1, H)
    w5_p = jnp.pad(W5, ((0, 0), (0, H - 1)))
    b5_p = jnp.pad(b5, (0, H - 1)).reshape(1, H)

    deg = _deg_kernel()(dsts, zeros_deg, ones_deg)
    g1, dinv_col = _tc_prep(x_p, W1, deg[0], deg[1])

    s1 = _agg_kernel()(g1, srcs, dsts, zeros_agg)
    r1, ssum1, ssq1 = _tc_epi1((s1[0], s1[1]), g1, dinv_col, b1r)

    g2a, g2b = _tc_bnmm(r1, ssum1, ssq1, g1r, be1r, w2a, w2b, dinv_col)

    s2a = _agg_kernel()(g2a, srcs, dsts, zeros_agg)
    s2b = _agg_kernel()(g2b, srcs, dsts, zeros_agg)
    r2, ssum2, ssq2 = _tc_epi2((s2a[0], s2a[1]), (s2b[0], s2b[1]),
                               g2a, g2b, dinv_col, b2a, b2b)
    h2a, h2b = _tc_bn2(r2, ssum2, ssq2, g2r, be2r)

    seg_a, seg_b, cnt = _pool_kernel()(h2a, h2b, batch_p, zseg, zcnt,
                                       ones_pool)

    out = _tc_head(seg_a[0, :B], seg_b[0, :B], seg_a[1, :B], seg_b[1, :B],
                   cnt[0, :B], cnt[1, :B], rdk_p, w3ea, w3eb, w3r,
                   b3r, g3r, be3r, W4, b4r, g4r, be4r, w5_p, b5_p)
    return out[:, 0]


# speedup vs baseline: 17.9951x; 1.1158x over previous
import functools

import jax
import jax.numpy as jnp
from jax import lax
from jax.experimental import pallas as pl
from jax.experimental.pallas import tpu as pltpu
from jax.experimental.pallas import tpu_sc as plsc

F32 = jnp.float32
I32 = jnp.int32

N = 10000
E = 320000
D = 128
B = 256
R = 200
H = 128

NP = 10240
NC, NS = 2, 16
NW = NC * NS
CHUNKS = 128
K = 80
NB = 4
EPW = CHUNKS * K
EP = EPW * NW
PAD_ROW = NP - 1
DW = 128
ZR = NP // NS

PRW = NP // NW
PC, PK = 5, 64
SEG = 384
PAD_SEG = SEG - 1

BR = 512
GRID = NP // BR
EPS = 1e-5


def _sc_mesh():
    return plsc.VectorSubcoreMesh(
        core_axis_name="c", subcore_axis_name="s",
        num_cores=NC, num_subcores=NS)


@functools.cache
def _deg_kernel():
    @functools.partial(
        pl.kernel,
        out_type=jax.ShapeDtypeStruct((NC, NP, DW), F32),
        mesh=_sc_mesh(),
        scratch_types=[
            pltpu.VMEM((CHUNKS, K), I32),
            pltpu.VMEM((16, DW), F32),
            pltpu.VMEM_SHARED((NP, DW), F32),
            pltpu.SemaphoreType.DMA,
        ],
    )
    def deg(dsts_hbm, zeros_hbm, ones_hbm, out_hbm, idx_d, ones_v, acc, sem):
        c = lax.axis_index("c")
        s = lax.axis_index("s")
        pltpu.sync_copy(zeros_hbm, acc.at[pl.ds(s * ZR, ZR)])
        pltpu.sync_copy(dsts_hbm.at[c, s], idx_d)
        pltpu.sync_copy(ones_hbm, ones_v)
        plsc.subcore_barrier()

        def chunk(j, carry):
            ivs = [idx_d[j, pl.ds(t * 16, 16)] for t in range(K // 16)]
            for t in range(K // 16):
                pltpu.async_copy(ones_v, acc.at[ivs[t]], sem, add=True)
            for t in range(K // 16):
                pltpu.make_async_copy(ones_v, acc.at[ivs[t]], sem).wait()
            return carry

        lax.fori_loop(0, CHUNKS, chunk, 0)
        plsc.subcore_barrier()
        pltpu.sync_copy(acc.at[pl.ds(s * ZR, ZR)],
                        out_hbm.at[c, pl.ds(s * ZR, ZR)])

    return deg


@functools.cache
def _agg_kernel():
    @functools.partial(
        pl.kernel,
        out_type=jax.ShapeDtypeStruct((NC, NP, D), F32),
        mesh=_sc_mesh(),
        scratch_types=(
            [pltpu.VMEM((K,), I32)] * (2 * NB)
            + [pltpu.VMEM((K, D), F32)] * NB
            + [pltpu.VMEM_SHARED((NP, D), F32)]
            + [pltpu.SemaphoreType.DMA] * (3 * NB)
        ),
    )
    def agg(g_hbm, srcs_hbm, dsts_hbm, zeros_hbm, out_hbm, *scr):
        idx_s = scr[0:NB]
        idx_d = scr[NB:2 * NB]
        rows = scr[2 * NB:3 * NB]
        acc = scr[3 * NB]
        sem_g = scr[3 * NB + 1:3 * NB + 1 + NB]
        sem_s = scr[3 * NB + 1 + NB:3 * NB + 1 + 2 * NB]
        sem_i = scr[3 * NB + 1 + 2 * NB:3 * NB + 1 + 3 * NB]
        c = lax.axis_index("c")
        s = lax.axis_index("s")
        pltpu.sync_copy(zeros_hbm, acc.at[pl.ds(s * ZR, ZR)])
        plsc.subcore_barrier()

        for b in range(NB):
            pltpu.sync_copy(srcs_hbm.at[c, s, b], idx_s[b])
            pltpu.sync_copy(dsts_hbm.at[c, s, b], idx_d[b])
            pltpu.async_copy(g_hbm.at[idx_s[b]], rows[b], sem_g[b])

        def _scatter_granules(b):
            ivs = [idx_d[b][pl.ds(t * 16, 16)] for t in range(K // 16)]
            for t in range(K // 16):
                pltpu.async_copy(rows[b].at[pl.ds(t * 16, 16)],
                                 acc.at[ivs[t]], sem_s[b], add=True)
            return ivs

        def _drain_granules(b, ivs):
            for t in range(K // 16):
                pltpu.make_async_copy(rows[b].at[pl.ds(t * 16, 16)],
                                      acc.at[ivs[t]], sem_s[b]).wait()

        def group(i, carry):
            not_last = i < CHUNKS // NB - 1
            ivs_all = []
            for b in range(NB):
                pltpu.make_async_copy(g_hbm.at[idx_s[b]], rows[b],
                                      sem_g[b]).wait()
                ivs_all.append(_scatter_granules(b))

                @pl.when(not_last)
                def _(b=b):
                    pltpu.async_copy(srcs_hbm.at[c, s, NB * i + NB + b],
                                     idx_s[b], sem_i[b])
                    pltpu.async_copy(dsts_hbm.at[c, s, NB * i + NB + b],
                                     idx_d[b], sem_i[b])

            @pl.when(not_last)
            def _():
                for b in range(NB):
                    _drain_granules(b, ivs_all[b])
                    pltpu.make_async_copy(srcs_hbm.at[c, s, 0], idx_s[b],
                                          sem_i[b]).wait()
                    pltpu.make_async_copy(dsts_hbm.at[c, s, 0], idx_d[b],
                                          sem_i[b]).wait()
                    pltpu.async_copy(g_hbm.at[idx_s[b]], rows[b], sem_g[b])

            return carry

        lax.fori_loop(0, CHUNKS // NB, group, 0)
        for b in range(NB):
            _drain_granules(b, [idx_d[b][pl.ds(t * 16, 16)]
                                for t in range(K // 16)])
        plsc.subcore_barrier()
        pltpu.sync_copy(acc.at[pl.ds(s * ZR, ZR)],
                        out_hbm.at[c, pl.ds(s * ZR, ZR)])

    return agg


@functools.cache
def _pool_kernel():
    @functools.partial(
        pl.kernel,
        out_type=(jax.ShapeDtypeStruct((NC, SEG, H), F32),
                  jax.ShapeDtypeStruct((NC, SEG, H), F32),
                  jax.ShapeDtypeStruct((NC, SEG, DW), F32)),
        mesh=_sc_mesh(),
        scratch_types=[
            pltpu.VMEM((PK,), I32),
            pltpu.VMEM((PK, H), F32),
            pltpu.VMEM((PK, H), F32),
            pltpu.VMEM((16, DW), F32),
            pltpu.VMEM_SHARED((SEG, H), F32),
            pltpu.VMEM_SHARED((SEG, H), F32),
            pltpu.VMEM_SHARED((SEG, DW), F32),
        ],
    )
    def pool(h2a_hbm, h2b_hbm, batch_hbm, zseg_hbm, zcnt_hbm, ones_hbm,
             sega_hbm, segb_hbm, cnt_hbm, idx_b, rows_a, rows_b, ones_v,
             acc_a, acc_b, acc_cnt):
        c = lax.axis_index("c")
        s = lax.axis_index("s")
        zr = SEG // NS
        pltpu.sync_copy(zseg_hbm, acc_a.at[pl.ds(s * zr, zr)])
        pltpu.sync_copy(zseg_hbm, acc_b.at[pl.ds(s * zr, zr)])
        pltpu.sync_copy(zcnt_hbm, acc_cnt.at[pl.ds(s * zr, zr)])
        pltpu.sync_copy(ones_hbm, ones_v)
        plsc.subcore_barrier()
        base = (c * NS + s) * PRW

        def chunk(j, carry):
            pltpu.sync_copy(batch_hbm.at[c, s, j], idx_b)
            pltpu.sync_copy(h2a_hbm.at[pl.ds(base + j * PK, PK)], rows_a)
            pltpu.sync_copy(h2b_hbm.at[pl.ds(base + j * PK, PK)], rows_b)
            for t in range(PK // 16):
                iv = idx_b[pl.ds(t * 16, 16)]
                pltpu.sync_copy(rows_a.at[pl.ds(t * 16, 16)], acc_a.at[iv],
                                add=True)
                pltpu.sync_copy(rows_b.at[pl.ds(t * 16, 16)], acc_b.at[iv],
                                add=True)
                pltpu.sync_copy(ones_v, acc_cnt.at[iv], add=True)
            return carry

        lax.fori_loop(0, PC, chunk, 0)
        plsc.subcore_barrier()
        pltpu.sync_copy(acc_a.at[pl.ds(s * zr, zr)],
                        sega_hbm.at[c, pl.ds(s * zr, zr)])
        pltpu.sync_copy(acc_b.at[pl.ds(s * zr, zr)],
                        segb_hbm.at[c, pl.ds(s * zr, zr)])
        pltpu.sync_copy(acc_cnt.at[pl.ds(s * zr, zr)],
                        cnt_hbm.at[c, pl.ds(s * zr, zr)])

    return pool


def _row_spec(w):
    return pl.BlockSpec((BR, w), lambda i: (i, 0))


def _whole(shape):
    return pl.BlockSpec(shape, lambda i: tuple(0 for _ in shape))


def _prep_body(x_ref, w1_ref, d0_ref, d1_ref, g1_ref, dinv_ref):
    d = d0_ref[:, 0:1] + d1_ref[:, 0:1] + 1.0
    dinv = jnp.broadcast_to(lax.rsqrt(d), (BR, D))
    xw = jnp.dot(x_ref[...], w1_ref[...], preferred_element_type=F32)
    g1_ref[...] = xw * dinv
    dinv_ref[...] = dinv


def _tc_prep(x_p, w1, deg0, deg1):
    return pl.pallas_call(
        _prep_body,
        grid=(GRID,),
        in_specs=[_row_spec(D), _whole((D, H)), _row_spec(DW), _row_spec(DW)],
        out_specs=[_row_spec(H), _row_spec(D)],
        out_shape=[jax.ShapeDtypeStruct((NP, H), F32),
                   jax.ShapeDtypeStruct((NP, D), F32)],
    )(x_p, w1, deg0, deg1)


def _epi1_body(s0_ref, s1_ref, g1_ref, dinv_ref, b1_ref,
               r1_ref, sum_ref, sq_ref):
    i = pl.program_id(0)
    r = jnp.maximum(
        dinv_ref[...] * (s0_ref[...] + s1_ref[...] + g1_ref[...])
        + b1_ref[...], 0.0)
    r1_ref[...] = r
    rowid = lax.broadcasted_iota(I32, (BR, H), 0) + i * BR
    rm = jnp.where(rowid < N, r, 0.0)

    @pl.when(i == 0)
    def _():
        sum_ref[...] = jnp.zeros((8, H), F32)
        sq_ref[...] = jnp.zeros((8, H), F32)

    sum_ref[0:1, :] += jnp.sum(rm, axis=0, keepdims=True)
    sq_ref[0:1, :] += jnp.sum(rm * rm, axis=0, keepdims=True)


def _tc_epi1(s1_parts, g1, dinv_col, b1):
    return pl.pallas_call(
        _epi1_body,
        grid=(GRID,),
        in_specs=[_row_spec(H), _row_spec(H), _row_spec(H), _row_spec(D),
                  _whole((1, H))],
        out_specs=[_row_spec(H), _whole((8, H)), _whole((8, H))],
        out_shape=[jax.ShapeDtypeStruct((NP, H), F32),
                   jax.ShapeDtypeStruct((8, H), F32),
                   jax.ShapeDtypeStruct((8, H), F32)],
    )(s1_parts[0], s1_parts[1], g1, dinv_col, b1)


def _bnmm_body(r1_ref, sum_ref, sq_ref, ga_ref, be_ref, w2a_ref, w2b_ref,
               dinv_ref, g2a_ref, g2b_ref):
    mu = sum_ref[0:1, :] * (1.0 / N)
    var = sq_ref[0:1, :] * (1.0 / N) - mu * mu
    sfac = ga_ref[...] * lax.rsqrt(var + EPS)
    t = be_ref[...] - mu * sfac
    h = r1_ref[...] * sfac + t
    dv = dinv_ref[...]
    g2a_ref[...] = jnp.dot(h, w2a_ref[...], preferred_element_type=F32) * dv
    g2b_ref[...] = jnp.dot(h, w2b_ref[...], preferred_element_type=F32) * dv


def _tc_bnmm(r1, ssum, ssq, gamma1, beta1, w2a, w2b, dinv_col):
    return pl.pallas_call(
        _bnmm_body,
        grid=(GRID,),
        in_specs=[_row_spec(H), _whole((8, H)), _whole((8, H)),
                  _whole((1, H)), _whole((1, H)),
                  _whole((H, H)), _whole((H, H)), _row_spec(D)],
        out_specs=[_row_spec(H), _row_spec(H)],
        out_shape=[jax.ShapeDtypeStruct((NP, H), F32),
                   jax.ShapeDtypeStruct((NP, H), F32)],
    )(r1, ssum, ssq, gamma1, beta1, w2a, w2b, dinv_col)


def _epi2_body(sa0_ref, sa1_ref, sb0_ref, sb1_ref, g2a_ref, g2b_ref,
               dinv_ref, b2a_ref, b2b_ref, r2_ref, sum_ref, sq_ref):
    i = pl.program_id(0)
    dv = dinv_ref[...]
    ra = jnp.maximum(
        dv * (sa0_ref[...] + sa1_ref[...] + g2a_ref[...]) + b2a_ref[...], 0.0)
    rb = jnp.maximum(
        dv * (sb0_ref[...] + sb1_ref[...] + g2b_ref[...]) + b2b_ref[...], 0.0)
    r2_ref[:, 0:H] = ra
    r2_ref[:, H:2 * H] = rb
    rowid = lax.broadcasted_iota(I32, (BR, H), 0) + i * BR
    ram = jnp.where(rowid < N, ra, 0.0)
    rbm = jnp.where(rowid < N, rb, 0.0)

    @pl.when(i == 0)
    def _():
        sum_ref[...] = jnp.zeros((8, 2 * H), F32)
        sq_ref[...] = jnp.zeros((8, 2 * H), F32)

    sum_ref[0:1, 0:H] += jnp.sum(ram, axis=0, keepdims=True)
    sum_ref[0:1, H:2 * H] += jnp.sum(rbm, axis=0, keepdims=True)
    sq_ref[0:1, 0:H] += jnp.sum(ram * ram, axis=0, keepdims=True)
    sq_ref[0:1, H:2 * H] += jnp.sum(rbm * rbm, axis=0, keepdims=True)


def _tc_epi2(s2a, s2b, g2a, g2b, dinv_col, b2a, b2b):
    return pl.pallas_call(
        _epi2_body,
        grid=(GRID,),
        in_specs=[_row_spec(H), _row_spec(H), _row_spec(H), _row_spec(H),
                  _row_spec(H), _row_spec(H), _row_spec(D),
                  _whole((1, H)), _whole((1, H))],
        out_specs=[_row_spec(2 * H), _whole((8, 2 * H)), _whole((8, 2 * H))],
        out_shape=[jax.ShapeDtypeStruct((NP, 2 * H), F32),
                   jax.ShapeDtypeStruct((8, 2 * H), F32),
                   jax.ShapeDtypeStruct((8, 2 * H), F32)],
    )(s2a[0], s2a[1], s2b[0], s2b[1], g2a, g2b, dinv_col, b2a, b2b)


def _bn2_body(r2_ref, sum_ref, sq_ref, ga_ref, be_ref, h2a_ref, h2b_ref):
    mu = sum_ref[0:1, :] * (1.0 / N)
    var = sq_ref[0:1, :] * (1.0 / N) - mu * mu
    sfac = ga_ref[...] * lax.rsqrt(var + EPS)
    t = be_ref[...] - mu * sfac
    h2 = r2_ref[...] * sfac + t
    h2a_ref[...] = h2[:, 0:H]
    h2b_ref[...] = h2[:, H:2 * H]


def _tc_bn2(r2, ssum, ssq, gamma2, beta2):
    return pl.pallas_call(
        _bn2_body,
        grid=(GRID,),
        in_specs=[_row_spec(2 * H), _whole((8, 2 * H)), _whole((8, 2 * H)),
                  _whole((1, 2 * H)), _whole((1, 2 * H))],
        out_specs=[_row_spec(H), _row_spec(H)],
        out_shape=[jax.ShapeDtypeStruct((NP, H), F32),
                   jax.ShapeDtypeStruct((NP, H), F32)],
    )(r2, ssum, ssq, gamma2, beta2)


def _bn_rows(z, gamma, beta):
    mu = jnp.mean(z, axis=0, keepdims=True)
    var = jnp.mean(z * z, axis=0, keepdims=True) - mu * mu
    return (z - mu) * lax.rsqrt(var + EPS) * gamma + beta


def _head_body(s0a_ref, s0b_ref, s1a_ref, s1b_ref, cnt0_ref, cnt1_ref,
               rdk_ref, w3ea_ref, w3eb_ref, w3r_ref, b3_ref, g3_ref, be3_ref,
               w4_ref, b4_ref, g4_ref, be4_ref, w5_ref, b5_ref, out_ref):
    cnt = cnt0_ref[:, 0:1] + cnt1_ref[:, 0:1]
    inv = 1.0 / jnp.maximum(cnt, 1.0)
    emb_a = (s0a_ref[...] + s1a_ref[...]) * inv
    emb_b = (s0b_ref[...] + s1b_ref[...]) * inv
    z = jnp.dot(emb_a, w3ea_ref[...], preferred_element_type=F32)
    z += jnp.dot(emb_b, w3eb_ref[...], preferred_element_type=F32)
    z += jnp.dot(rdk_ref[...], w3r_ref[...], preferred_element_type=F32)
    z = jnp.maximum(z + b3_ref[...], 0.0)
    z = _bn_rows(z, g3_ref[...], be3_ref[...])
    z = jnp.maximum(
        jnp.dot(z, w4_ref[...], preferred_element_type=F32) + b4_ref[...], 0.0)
    z = _bn_rows(z, g4_ref[...], be4_ref[...])
    out_ref[...] = (jnp.dot(z, w5_ref[...], preferred_element_type=F32)
                    + b5_ref[...])


def _tc_head(s0a, s0b, s1a, s1b, cnt0, cnt1, rdk_p, w3ea, w3eb, w3r,
             b3, g3, be3, w4, b4, g4, be4, w5_p, b5_p):
    shapes = [(B, H), (B, H), (B, H), (B, H), (B, DW), (B, DW), (B, 2 * H),
              (H, 2 * H), (H, 2 * H), (2 * H, 2 * H), (1, 2 * H), (1, 2 * H),
              (1, 2 * H), (2 * H, H), (1, H), (1, H), (1, H),
              (H, H), (1, H)]
    return pl.pallas_call(
        _head_body,
        grid=(1,),
        in_specs=[_whole(s) for s in shapes],
        out_specs=_whole((B, H)),
        out_shape=jax.ShapeDtypeStruct((B, H), F32),
    )(s0a, s0b, s1a, s1b, cnt0, cnt1, rdk_p, w3ea, w3eb, w3r,
      b3, g3, be3, w4, b4, g4, be4, w5_p, b5_p)


def kernel(x, edge_index, batch, rdkit_feats, W1, b1, gamma1, beta1,
           W2, b2, gamma2, beta2, W3, b3, gamma3, beta3,
           W4, b4, gamma4, beta4, W5, b5):
    ei = edge_index.astype(I32)
    pad_e = N + (jnp.arange(EP - E, dtype=I32) % (NP - N))
    srcs = jnp.concatenate([ei[0], pad_e]).reshape(NC, NS, CHUNKS, K)
    dsts = jnp.concatenate([ei[1], pad_e]).reshape(NC, NS, CHUNKS, K)
    x_p = jnp.pad(x, ((0, NP - N), (0, 0)))
    batch_p = jnp.concatenate(
        [batch.astype(I32),
         B + (jnp.arange(NP - N, dtype=I32) % (SEG - B))]
    ).reshape(NC, NS, PC, PK)
    rdk_p = jnp.pad(rdkit_feats, ((0, 0), (0, 2 * H - R)))

    zeros_deg = jnp.zeros((ZR, DW), F32)
    ones_deg = jnp.ones((16, DW), F32)
    zeros_agg = jnp.zeros((ZR, D), F32)
    zseg = jnp.zeros((SEG // NS, H), F32)
    zcnt = jnp.zeros((SEG // NS, DW), F32)
    ones_pool = jnp.ones((16, DW), F32)

    b1r = b1.reshape(1, H)
    g1r, be1r = gamma1.reshape(1, H), beta1.reshape(1, H)
    w2a, w2b = W2[:, :H], W2[:, H:]
    b2a, b2b = b2[:H].reshape(1, H), b2[H:].reshape(1, H)
    g2r, be2r = gamma2.reshape(1, 2 * H), beta2.reshape(1, 2 * H)
    w3ea, w3eb = W3[:H], W3[H:2 * H]
    w3r = jnp.pad(W3[2 * H:], ((0, 2 * H - R), (0, 0)))
    b3r = b3.reshape(1, 2 * H)
    g3r, be3r = gamma3.reshape(1, 2 * H), beta3.reshape(1, 2 * H)
    b4r = b4.reshape(1, H)
    g4r, be4r = gamma4.reshape(1, H), beta4.reshape(1, H)
    w5_p = jnp.pad(W5, ((0, 0), (0, H - 1)))
    b5_p = jnp.pad(b5, (0, H - 1)).reshape(1, H)

    deg = _deg_kernel()(dsts, zeros_deg, ones_deg)
    g1, dinv_col = _tc_prep(x_p, W1, deg[0], deg[1])

    s1 = _agg_kernel()(g1, srcs, dsts, zeros_agg)
    r1, ssum1, ssq1 = _tc_epi1((s1[0], s1[1]), g1, dinv_col, b1r)

    g2a, g2b = _tc_bnmm(r1, ssum1, ssq1, g1r, be1r, w2a, w2b, dinv_col)

    s2a = _agg_kernel()(g2a, srcs, dsts, zeros_agg)
    s2b = _agg_kernel()(g2b, srcs, dsts, zeros_agg)
    r2, ssum2, ssq2 = _tc_epi2((s2a[0], s2a[1]), (s2b[0], s2b[1]),
                               g2a, g2b, dinv_col, b2a, b2b)
    h2a, h2b = _tc_bn2(r2, ssum2, ssq2, g2r, be2r)

    seg_a, seg_b, cnt = _pool_kernel()(h2a, h2b, batch_p, zseg, zcnt,
                                       ones_pool)

    out = _tc_head(seg_a[0, :B], seg_b[0, :B], seg_a[1, :B], seg_b[1, :B],
                   cnt[0, :B], cnt[1, :B], rdk_p, w3ea, w3eb, w3r,
                   b3r, g3r, be3r, W4, b4r, g4r, be4r, w5_p, b5_p)
    return out[:, 0]
